# baseline (speedup 1.0000x reference)
import jax
import jax.numpy as jnp
from jax import lax
from jax.experimental import pallas as pl
from jax.experimental.pallas import tpu as pltpu
from jax.experimental.pallas import tpu_sc as plsc

N_WORD = 40000
N_TOPIC = 800
N_DOC = 3200
B = 16
H = 128
VOCAB = 15000
NUM_TOPIC = 50

OW = 40960
OT = 1024
ND = 3328
E_WD = 200000
E_TD = 20000
CW = 12544
CT = 1280
EW = 6272
ET = 640
NW_CH = 2560
NT_CH = 64


def _rsqrt16(x):
    i = lax.bitcast_convert_type(x, jnp.int32)
    i = jnp.int32(0x5F3759DF) - lax.shift_right_arithmetic(i, 1)
    y = lax.bitcast_convert_type(i, jnp.float32)
    for _ in range(2):
        y = y * (1.5 - 0.5 * x * y * y)
    return y


def _fill_i32(buf, start, count, value):
    v = jnp.full((16,), value, jnp.int32)

    def bd(i, _):
        buf[pl.ds(start + i * 16, 16)] = v
        return 0
    lax.fori_loop(0, count // 16, bd, 0)


def _k1_body(we_ref, te_ref, wwd_ref, wtd_ref, ow_ref, uws_ref, uts_ref):
    ow = ow_ref[...]
    vw = jnp.dot(wwd_ref[...], ow, preferred_element_type=jnp.float32)
    uws_ref[...] = jnp.dot(we_ref[...], vw, preferred_element_type=jnp.float32)[:, 0]

    @pl.when(pl.program_id(0) == 0)
    def _():
        vt = jnp.dot(wtd_ref[...], ow, preferred_element_type=jnp.float32)
        uts_ref[...] = jnp.dot(te_ref[...], vt, preferred_element_type=jnp.float32)[:, 0]


def _sc_body(wd_src, wd_dst, td_src, td_dst, word_ids, topic_ids,
             uws, uts, ones_hbm, zeros_hbm, out,
             cnt_ow, cnt_iw, cnt_ot, cnt_it, m_wd, m_td, nv_w, nv_t,
             zbuf, ones, ones_t,
             ibufS, ibufD, tbufS, tbufD, jbufS, jbufD, kbufS, kbufD,
             vbuf, vbuf_t, gidx, gbuf, cbuf, nbuf,
             gidx_t, gbuf_t, cbuf_t, nbuf_t,
             mw, mt, ciw, cit, accbuf,
             semA, semB, semC, semD, semE):
    c = lax.axis_index("c")
    s = lax.axis_index("s")
    w = c * 16 + s

    d_ones = pltpu.async_copy(ones_hbm, ones, semA)
    d_onest = pltpu.async_copy(ones_hbm.at[pl.ds(0, CT)], ones_t, semA)
    d_zbuf = pltpu.async_copy(zeros_hbm, zbuf, semB)
    o_cw = jnp.where(s == 15, E_WD - CW, s * CW)
    d_is = pltpu.async_copy(wd_src.at[pl.ds(o_cw, CW)], ibufS, semC)
    d_id = pltpu.async_copy(wd_dst.at[pl.ds(o_cw, CW)], ibufD, semC)
    o_ct = jnp.where(s == 15, E_TD - CT, s * CT)
    d_ts = pltpu.async_copy(td_src.at[pl.ds(o_ct, CT)], tbufS, semC)
    d_td = pltpu.async_copy(td_dst.at[pl.ds(o_ct, CT)], tbufD, semC)
    o_bw = jnp.where(w == 31, E_WD - EW, w * EW)
    d_js = pltpu.async_copy(wd_src.at[pl.ds(o_bw, EW)], jbufS, semD)
    d_jd = pltpu.async_copy(wd_dst.at[pl.ds(o_bw, EW)], jbufD, semD)
    o_bt = jnp.where(w == 31, E_TD - ET, w * ET)
    d_ks = pltpu.async_copy(td_src.at[pl.ds(o_bt, ET)], kbufS, semD)
    d_kd = pltpu.async_copy(td_dst.at[pl.ds(o_bt, ET)], kbufD, semD)
    o_nw = jnp.where(s == 15, N_WORD - NW_CH, s * NW_CH)
    d_gi = pltpu.async_copy(word_ids.at[pl.ds(o_nw, NW_CH)], gidx, semE)
    o_nt = jnp.minimum(s * NT_CH, N_TOPIC - NT_CH)
    d_gt = pltpu.async_copy(topic_ids.at[pl.ds(o_nt, NT_CH)], gidx_t, semE)

    d_zbuf.wait()
    z1 = pltpu.async_copy(zbuf, cnt_ow.at[pl.ds(s * 2560, 2560)], semB)
    z2 = pltpu.async_copy(zbuf.at[pl.ds(0, 208)], cnt_iw.at[pl.ds(s * 208, 208)], semB)
    z3 = pltpu.async_copy(zbuf.at[pl.ds(0, 64)], cnt_ot.at[pl.ds(s * 64, 64)], semB)
    z4 = pltpu.async_copy(zbuf.at[pl.ds(0, 208)], cnt_it.at[pl.ds(s * 208, 208)], semB)
    z5 = pltpu.async_copy(zbuf.at[pl.ds(0, 208)], m_wd.at[pl.ds(s * 208, 208)], semB)
    z6 = pltpu.async_copy(zbuf.at[pl.ds(0, 208)], m_td.at[pl.ds(s * 208, 208)], semB)
    for z in (z1, z2, z3, z4, z5, z6):
        z.wait()

    d_gi.wait()
    d_gt.wait()
    d_uw = pltpu.async_copy(uws.at[gidx], gbuf, semE)
    d_ut = pltpu.async_copy(uts.at[gidx_t], gbuf_t, semE)
    plsc.subcore_barrier()

    for d in (d_is, d_id, d_ts, d_td, d_ones, d_onest):
        d.wait()

    @pl.when(s == 15)
    def _():
        _fill_i32(ibufS, 0, 704, OW - 1)
        _fill_i32(ibufD, 0, 704, ND - 1)
        _fill_i32(tbufS, 0, 480, OT - 1)
        _fill_i32(tbufD, 0, 480, ND - 1)

    s1 = pltpu.async_copy(ones, cnt_ow.at[ibufS], semC, add=True)
    s3 = pltpu.async_copy(ones_t, cnt_ot.at[tbufS], semC, add=True)
    s1.wait()
    s3.wait()
    plsc.subcore_barrier()

    s2 = pltpu.async_copy(ones, cnt_iw.at[ibufD], semC, add=True)
    s4 = pltpu.async_copy(ones_t, cnt_it.at[tbufD], semC, add=True)

    d_cb = pltpu.async_copy(cnt_ow.at[pl.ds(o_nw, NW_CH)], cbuf, semB)
    d_cbt = pltpu.async_copy(cnt_ot.at[pl.ds(o_nt, NT_CH)], cbuf_t, semB)
    d_uw.wait()
    d_ut.wait()
    d_cb.wait()
    d_cbt.wait()

    def _nv16(i, _):
        cnt = jnp.maximum(cbuf[pl.ds(i * 16, 16)], 1.0)
        nbuf[pl.ds(i * 16, 16)] = _rsqrt16(cnt) * gbuf[pl.ds(i * 16, 16)]
        return 0
    lax.fori_loop(0, NW_CH // 16, _nv16, 0)

    def _nvt16(i, _):
        cnt = jnp.maximum(cbuf_t[pl.ds(i * 16, 16)], 1.0)
        nbuf_t[pl.ds(i * 16, 16)] = _rsqrt16(cnt) * gbuf_t[pl.ds(i * 16, 16)]
        return 0
    lax.fori_loop(0, NT_CH // 16, _nvt16, 0)
    d_nv = pltpu.async_copy(nbuf, nv_w.at[pl.ds(o_nw, NW_CH)], semB)
    d_nvt = pltpu.async_copy(nbuf_t, nv_t.at[pl.ds(o_nt, NT_CH)], semB)
    d_nv.wait()
    d_nvt.wait()
    plsc.subcore_barrier()

    for d in (d_js, d_jd, d_ks, d_kd):
        d.wait()

    @pl.when(w == 31)
    def _():
        _fill_i32(jbufS, 0, 704, OW - 1)
        _fill_i32(jbufD, 0, 704, ND - 1)
        _fill_i32(kbufS, 0, 480, OT - 1)
        _fill_i32(kbufD, 0, 480, ND - 1)

    g1 = pltpu.async_copy(nv_w.at[jbufS], vbuf, semD)
    g2 = pltpu.async_copy(nv_t.at[kbufS], vbuf_t, semD)
    g1.wait()
    g2.wait()
    x1 = pltpu.async_copy(vbuf, m_wd.at[jbufD], semD, add=True)
    x2 = pltpu.async_copy(vbuf_t, m_td.at[kbufD], semD, add=True)
    x1.wait()
    x2.wait()
    s2.wait()
    s4.wait()
    plsc.subcore_barrier()

    base = s * 200
    r1 = pltpu.async_copy(m_wd.at[pl.ds(base, 208)], mw, semB)
    r2 = pltpu.async_copy(m_td.at[pl.ds(base, 208)], mt, semB)
    r3 = pltpu.async_copy(cnt_iw.at[pl.ds(base, 208)], ciw, semB)
    r4 = pltpu.async_copy(cnt_it.at[pl.ds(base, 208)], cit, semB)
    for d in (r1, r2, r3, r4):
        d.wait()
    lane = lax.iota(jnp.int32, 16)

    def _dot16(j, acc):
        cw = jnp.maximum(ciw[pl.ds(j * 16, 16)], 1.0)
        ct = jnp.maximum(cit[pl.ds(j * 16, 16)], 1.0)
        v = mw[pl.ds(j * 16, 16)] * _rsqrt16(cw) + mt[pl.ds(j * 16, 16)] * _rsqrt16(ct)
        return acc + jnp.where(j * 16 + lane < 200, v, 0.0)
    acc = lax.fori_loop(0, 13, _dot16, jnp.zeros((16,), jnp.float32))
    accbuf[...] = acc
    pltpu.sync_copy(accbuf, out.at[pl.ds(w * 16, 16)])


def _k3_body(p_ref, y_ref, bw_ref, bt_ref, ow_ref, ob_ref, loss_ref, pred_ref):
    dsum = jnp.sum(p_ref[...], axis=0, keepdims=True)
    bias = jnp.sum((bw_ref[...] + bt_ref[...]) * ow_ref[...].T) + ob_ref[0, 0]
    logits = dsum / 200.0 + bias
    y = y_ref[...]
    loss_ref[...] = jnp.mean(
        jnp.maximum(logits, 0.0) - logits * y
        + jnp.log(1.0 + jnp.exp(-jnp.abs(logits))), keepdims=True).reshape(1, 1)
    pred_ref[...] = 1.0 / (1.0 + jnp.exp(-logits))


def kernel(word_ids, topic_ids, wd_src, wd_dst, ww_src, ww_dst, wt_src, wt_dst,
           td_src, td_dst, tt_src, tt_dst, y_data, word_embeds, topic_embeds,
           W_wt, b_wt, W_ww, b_ww, W_wd, b_wd, W_td, b_td, W_tt, b_tt, out_W, out_b):
    f32, i32 = jnp.float32, jnp.int32

    uws, uts = pl.pallas_call(
        _k1_body,
        grid=(5,),
        in_specs=[pl.BlockSpec((3072, H), lambda i: (i, 0)),
                  pl.BlockSpec((NUM_TOPIC, H), lambda i: (0, 0)),
                  pl.BlockSpec((H, H), lambda i: (0, 0)),
                  pl.BlockSpec((H, H), lambda i: (0, 0)),
                  pl.BlockSpec((H, 1), lambda i: (0, 0))],
        out_specs=[pl.BlockSpec((3072,), lambda i: (i,)),
                   pl.BlockSpec((NUM_TOPIC,), lambda i: (0,))],
        out_shape=[jax.ShapeDtypeStruct((VOCAB,), f32),
                   jax.ShapeDtypeStruct((NUM_TOPIC,), f32)],
    )(word_embeds, topic_embeds, W_wd, W_td, out_W)

    ones_hbm = jnp.ones((CW,), f32)
    zeros_hbm = jnp.zeros((2560,), f32)

    mesh = plsc.VectorSubcoreMesh(core_axis_name="c", subcore_axis_name="s")
    partial = pl.kernel(
        _sc_body,
        out_type=jax.ShapeDtypeStruct((512,), f32),
        mesh=mesh,
        scratch_types=[
            pltpu.VMEM_SHARED((OW,), f32),
            pltpu.VMEM_SHARED((ND,), f32),
            pltpu.VMEM_SHARED((OT,), f32),
            pltpu.VMEM_SHARED((ND,), f32),
            pltpu.VMEM_SHARED((ND,), f32),
            pltpu.VMEM_SHARED((ND,), f32),
            pltpu.VMEM_SHARED((OW,), f32),
            pltpu.VMEM_SHARED((OT,), f32),
            pltpu.VMEM((2560,), f32),
            pltpu.VMEM((CW,), f32),
            pltpu.VMEM((CT,), f32),
            pltpu.VMEM((CW,), i32),
            pltpu.VMEM((CW,), i32),
            pltpu.VMEM((CT,), i32),
            pltpu.VMEM((CT,), i32),
            pltpu.VMEM((EW,), i32),
            pltpu.VMEM((EW,), i32),
            pltpu.VMEM((ET,), i32),
            pltpu.VMEM((ET,), i32),
            pltpu.VMEM((EW,), f32),
            pltpu.VMEM((ET,), f32),
            pltpu.VMEM((NW_CH,), i32),
            pltpu.VMEM((NW_CH,), f32),
            pltpu.VMEM((NW_CH,), f32),
            pltpu.VMEM((NW_CH,), f32),
            pltpu.VMEM((NT_CH,), i32),
            pltpu.VMEM((NT_CH,), f32),
            pltpu.VMEM((NT_CH,), f32),
            pltpu.VMEM((NT_CH,), f32),
            pltpu.VMEM((208,), f32),
            pltpu.VMEM((208,), f32),
            pltpu.VMEM((208,), f32),
            pltpu.VMEM((208,), f32),
            pltpu.VMEM((16,), f32),
            pltpu.SemaphoreType.DMA,
            pltpu.SemaphoreType.DMA,
            pltpu.SemaphoreType.DMA,
            pltpu.SemaphoreType.DMA,
            pltpu.SemaphoreType.DMA,
        ],
    )(wd_src, wd_dst, td_src, td_dst, word_ids, topic_ids,
      uws, uts, ones_hbm, zeros_hbm)

    loss, pred = pl.pallas_call(
        _k3_body,
        out_shape=[jax.ShapeDtypeStruct((1, 1), f32),
                   jax.ShapeDtypeStruct((1, B), f32)],
    )(partial.reshape(32, 16), y_data.reshape(1, B), b_wd.reshape(1, H),
      b_td.reshape(1, H), out_W, out_b.reshape(1, 1))

    return loss.reshape(()), pred.reshape(B, 1)


# speedup vs baseline: 1.0069x; 1.0069x over previous
import jax
import jax.numpy as jnp
from jax import lax
from jax.experimental import pallas as pl
from jax.experimental.pallas import tpu as pltpu
from jax.experimental.pallas import tpu_sc as plsc

N_WORD = 40000
N_TOPIC = 800
N_DOC = 3200
B = 16
H = 128
VOCAB = 15000
NUM_TOPIC = 50

OW = 40960
OT = 1024
ND = 3328
E_WD = 200000
E_TD = 20000
CW = 12544
CT = 1280
EW = 6272
ET = 640
NW_CH = 2560
NT_CH = 64


def _rsqrt16(x):
    i = lax.bitcast_convert_type(x, jnp.int32)
    i = jnp.int32(0x5F3759DF) - lax.shift_right_arithmetic(i, 1)
    y = lax.bitcast_convert_type(i, jnp.float32)
    for _ in range(2):
        y = y * (1.5 - 0.5 * x * y * y)
    return y


def _fill_i32(buf, start, count, value):
    v = jnp.full((16,), value, jnp.int32)

    def bd(i, _):
        buf[pl.ds(start + i * 16, 16)] = v
        return 0
    lax.fori_loop(0, count // 16, bd, 0)


def _k1_body(we_ref, te_ref, wwd_ref, wtd_ref, ow_ref, uws_ref, uts_ref):
    ow = ow_ref[...]
    vw = jnp.dot(wwd_ref[...], ow, preferred_element_type=jnp.float32)
    uws_ref[...] = jnp.dot(we_ref[...], vw, preferred_element_type=jnp.float32)[:, 0]

    @pl.when(pl.program_id(0) == 0)
    def _():
        vt = jnp.dot(wtd_ref[...], ow, preferred_element_type=jnp.float32)
        uts_ref[...] = jnp.dot(te_ref[...], vt, preferred_element_type=jnp.float32)[:, 0]


def _sc_body(wd_src, wd_dst, td_src, td_dst, word_ids, topic_ids,
             uws, uts, ones_hbm, zeros_hbm, out,
             cnt_ow, cnt_iw, cnt_ot, cnt_it, m_wd, m_td, nv_w, nv_t,
             zbuf, ones, ones_t,
             ibufS, ibufD, tbufS, tbufD, jbufS, jbufD1, jbufD2, kbufS, kbufD,
             vbufA, vbufB, vbuf_t, gidx, gbuf, cbuf, nbuf,
             gidx_t, gbuf_t, cbuf_t, nbuf_t,
             mw, mt, ciw, cit, accbuf,
             semA, semB, semC, semD, semE):
    c = lax.axis_index("c")
    s = lax.axis_index("s")
    w = c * 16 + s

    d_ones = pltpu.async_copy(ones_hbm, ones, semA)
    d_onest = pltpu.async_copy(ones_hbm.at[pl.ds(0, CT)], ones_t, semA)
    d_zbuf = pltpu.async_copy(zeros_hbm, zbuf, semB)
    o_cw = jnp.where(s == 15, E_WD - CW, s * CW)
    d_is = pltpu.async_copy(wd_src.at[pl.ds(o_cw, CW)], ibufS, semC)
    d_id = pltpu.async_copy(wd_dst.at[pl.ds(o_cw, CW)], ibufD, semC)
    o_ct = jnp.where(s == 15, E_TD - CT, s * CT)
    d_ts = pltpu.async_copy(td_src.at[pl.ds(o_ct, CT)], tbufS, semC)
    d_td = pltpu.async_copy(td_dst.at[pl.ds(o_ct, CT)], tbufD, semC)
    o_bw = jnp.where(w == 31, E_WD - EW, w * EW)
    d_js = pltpu.async_copy(wd_src.at[pl.ds(o_bw, EW)], jbufS, semD)
    d_jd1 = pltpu.async_copy(wd_dst.at[pl.ds(o_bw, EW // 2)], jbufD1, semD)
    d_jd2 = pltpu.async_copy(wd_dst.at[pl.ds(o_bw + EW // 2, EW // 2)], jbufD2, semD)
    o_bt = jnp.where(w == 31, E_TD - ET, w * ET)
    d_ks = pltpu.async_copy(td_src.at[pl.ds(o_bt, ET)], kbufS, semD)
    d_kd = pltpu.async_copy(td_dst.at[pl.ds(o_bt, ET)], kbufD, semD)
    o_nw = jnp.where(s == 15, N_WORD - NW_CH, s * NW_CH)
    d_gi = pltpu.async_copy(word_ids.at[pl.ds(o_nw, NW_CH)], gidx, semE)
    o_nt = jnp.minimum(s * NT_CH, N_TOPIC - NT_CH)
    d_gt = pltpu.async_copy(topic_ids.at[pl.ds(o_nt, NT_CH)], gidx_t, semE)

    d_zbuf.wait()
    z1 = pltpu.async_copy(zbuf, cnt_ow.at[pl.ds(s * 2560, 2560)], semB)
    z2 = pltpu.async_copy(zbuf.at[pl.ds(0, 208)], cnt_iw.at[pl.ds(s * 208, 208)], semB)
    z3 = pltpu.async_copy(zbuf.at[pl.ds(0, 64)], cnt_ot.at[pl.ds(s * 64, 64)], semB)
    z4 = pltpu.async_copy(zbuf.at[pl.ds(0, 208)], cnt_it.at[pl.ds(s * 208, 208)], semB)
    z5 = pltpu.async_copy(zbuf.at[pl.ds(0, 208)], m_wd.at[pl.ds(s * 208, 208)], semB)
    z6 = pltpu.async_copy(zbuf.at[pl.ds(0, 208)], m_td.at[pl.ds(s * 208, 208)], semB)
    for z in (z1, z2, z3, z4, z5, z6):
        z.wait()

    d_gi.wait()
    d_gt.wait()
    d_uw = pltpu.async_copy(uws.at[gidx], gbuf, semE)
    d_ut = pltpu.async_copy(uts.at[gidx_t], gbuf_t, semE)
    plsc.subcore_barrier()

    for d in (d_is, d_id, d_ts, d_td, d_ones, d_onest):
        d.wait()

    @pl.when(s == 15)
    def _():
        _fill_i32(ibufS, 0, 704, OW - 1)
        _fill_i32(ibufD, 0, 704, ND - 1)
        _fill_i32(tbufS, 0, 480, OT - 1)
        _fill_i32(tbufD, 0, 480, ND - 1)

    s1 = pltpu.async_copy(ones, cnt_ow.at[ibufS], semC, add=True)
    s3 = pltpu.async_copy(ones_t, cnt_ot.at[tbufS], semC, add=True)
    s1.wait()
    s3.wait()
    plsc.subcore_barrier()

    s2 = pltpu.async_copy(ones, cnt_iw.at[ibufD], semC, add=True)
    s4 = pltpu.async_copy(ones_t, cnt_it.at[tbufD], semC, add=True)

    d_cb = pltpu.async_copy(cnt_ow.at[pl.ds(o_nw, NW_CH)], cbuf, semB)
    d_cbt = pltpu.async_copy(cnt_ot.at[pl.ds(o_nt, NT_CH)], cbuf_t, semB)
    d_uw.wait()
    d_ut.wait()
    d_cb.wait()
    d_cbt.wait()

    def _nv16(i, _):
        cnt = jnp.maximum(cbuf[pl.ds(i * 16, 16)], 1.0)
        nbuf[pl.ds(i * 16, 16)] = _rsqrt16(cnt) * gbuf[pl.ds(i * 16, 16)]
        return 0
    lax.fori_loop(0, NW_CH // 16, _nv16, 0)

    def _nvt16(i, _):
        cnt = jnp.maximum(cbuf_t[pl.ds(i * 16, 16)], 1.0)
        nbuf_t[pl.ds(i * 16, 16)] = _rsqrt16(cnt) * gbuf_t[pl.ds(i * 16, 16)]
        return 0
    lax.fori_loop(0, NT_CH // 16, _nvt16, 0)
    d_nv = pltpu.async_copy(nbuf, nv_w.at[pl.ds(o_nw, NW_CH)], semB)
    d_nvt = pltpu.async_copy(nbuf_t, nv_t.at[pl.ds(o_nt, NT_CH)], semB)
    d_nv.wait()
    d_nvt.wait()
    plsc.subcore_barrier()

    for d in (d_js, d_jd1, d_jd2, d_ks, d_kd):
        d.wait()

    @pl.when(w == 31)
    def _():
        _fill_i32(jbufS, 0, 704, OW - 1)
        _fill_i32(jbufD1, 0, 704, ND - 1)
        _fill_i32(kbufS, 0, 480, OT - 1)
        _fill_i32(kbufD, 0, 480, ND - 1)

    g1a = pltpu.async_copy(nv_w.at[jbufS.at[pl.ds(0, EW // 2)]], vbufA, semD)
    g1b = pltpu.async_copy(nv_w.at[jbufS.at[pl.ds(EW // 2, EW // 2)]], vbufB, semD)
    g2 = pltpu.async_copy(nv_t.at[kbufS], vbuf_t, semD)
    g1a.wait()
    x1a = pltpu.async_copy(vbufA, m_wd.at[jbufD1], semD, add=True)
    g1b.wait()
    x1b = pltpu.async_copy(vbufB, m_wd.at[jbufD2], semD, add=True)
    g2.wait()
    x2 = pltpu.async_copy(vbuf_t, m_td.at[kbufD], semD, add=True)
    x1a.wait()
    x1b.wait()
    x2.wait()
    s2.wait()
    s4.wait()
    plsc.subcore_barrier()

    base = s * 200
    r1 = pltpu.async_copy(m_wd.at[pl.ds(base, 208)], mw, semB)
    r2 = pltpu.async_copy(m_td.at[pl.ds(base, 208)], mt, semB)
    r3 = pltpu.async_copy(cnt_iw.at[pl.ds(base, 208)], ciw, semB)
    r4 = pltpu.async_copy(cnt_it.at[pl.ds(base, 208)], cit, semB)
    for d in (r1, r2, r3, r4):
        d.wait()
    lane = lax.iota(jnp.int32, 16)

    def _dot16(j, acc):
        cw = jnp.maximum(ciw[pl.ds(j * 16, 16)], 1.0)
        ct = jnp.maximum(cit[pl.ds(j * 16, 16)], 1.0)
        v = mw[pl.ds(j * 16, 16)] * _rsqrt16(cw) + mt[pl.ds(j * 16, 16)] * _rsqrt16(ct)
        return acc + jnp.where(j * 16 + lane < 200, v, 0.0)
    acc = lax.fori_loop(0, 13, _dot16, jnp.zeros((16,), jnp.float32))
    accbuf[...] = acc
    pltpu.sync_copy(accbuf, out.at[pl.ds(w * 16, 16)])


def _k3_body(p_ref, y_ref, bw_ref, bt_ref, ow_ref, ob_ref, loss_ref, pred_ref):
    dsum = jnp.sum(p_ref[...], axis=0, keepdims=True)
    bias = jnp.sum((bw_ref[...] + bt_ref[...]) * ow_ref[...].T) + ob_ref[0, 0]
    logits = dsum / 200.0 + bias
    y = y_ref[...]
    loss_ref[...] = jnp.mean(
        jnp.maximum(logits, 0.0) - logits * y
        + jnp.log(1.0 + jnp.exp(-jnp.abs(logits))), keepdims=True).reshape(1, 1)
    pred_ref[...] = 1.0 / (1.0 + jnp.exp(-logits))


def kernel(word_ids, topic_ids, wd_src, wd_dst, ww_src, ww_dst, wt_src, wt_dst,
           td_src, td_dst, tt_src, tt_dst, y_data, word_embeds, topic_embeds,
           W_wt, b_wt, W_ww, b_ww, W_wd, b_wd, W_td, b_td, W_tt, b_tt, out_W, out_b):
    f32, i32 = jnp.float32, jnp.int32

    uws, uts = pl.pallas_call(
        _k1_body,
        grid=(5,),
        in_specs=[pl.BlockSpec((3072, H), lambda i: (i, 0)),
                  pl.BlockSpec((NUM_TOPIC, H), lambda i: (0, 0)),
                  pl.BlockSpec((H, H), lambda i: (0, 0)),
                  pl.BlockSpec((H, H), lambda i: (0, 0)),
                  pl.BlockSpec((H, 1), lambda i: (0, 0))],
        out_specs=[pl.BlockSpec((3072,), lambda i: (i,)),
                   pl.BlockSpec((NUM_TOPIC,), lambda i: (0,))],
        out_shape=[jax.ShapeDtypeStruct((VOCAB,), f32),
                   jax.ShapeDtypeStruct((NUM_TOPIC,), f32)],
    )(word_embeds, topic_embeds, W_wd, W_td, out_W)

    ones_hbm = jnp.ones((CW,), f32)
    zeros_hbm = jnp.zeros((2560,), f32)

    mesh = plsc.VectorSubcoreMesh(core_axis_name="c", subcore_axis_name="s")
    partial = pl.kernel(
        _sc_body,
        out_type=jax.ShapeDtypeStruct((512,), f32),
        mesh=mesh,
        scratch_types=[
            pltpu.VMEM_SHARED((OW,), f32),
            pltpu.VMEM_SHARED((ND,), f32),
            pltpu.VMEM_SHARED((OT,), f32),
            pltpu.VMEM_SHARED((ND,), f32),
            pltpu.VMEM_SHARED((ND,), f32),
            pltpu.VMEM_SHARED((ND,), f32),
            pltpu.VMEM_SHARED((OW,), f32),
            pltpu.VMEM_SHARED((OT,), f32),
            pltpu.VMEM((2560,), f32),
            pltpu.VMEM((CW,), f32),
            pltpu.VMEM((CT,), f32),
            pltpu.VMEM((CW,), i32),
            pltpu.VMEM((CW,), i32),
            pltpu.VMEM((CT,), i32),
            pltpu.VMEM((CT,), i32),
            pltpu.VMEM((EW,), i32),
            pltpu.VMEM((EW // 2,), i32),
            pltpu.VMEM((EW // 2,), i32),
            pltpu.VMEM((ET,), i32),
            pltpu.VMEM((ET,), i32),
            pltpu.VMEM((EW // 2,), f32),
            pltpu.VMEM((EW // 2,), f32),
            pltpu.VMEM((ET,), f32),
            pltpu.VMEM((NW_CH,), i32),
            pltpu.VMEM((NW_CH,), f32),
            pltpu.VMEM((NW_CH,), f32),
            pltpu.VMEM((NW_CH,), f32),
            pltpu.VMEM((NT_CH,), i32),
            pltpu.VMEM((NT_CH,), f32),
            pltpu.VMEM((NT_CH,), f32),
            pltpu.VMEM((NT_CH,), f32),
            pltpu.VMEM((208,), f32),
            pltpu.VMEM((208,), f32),
            pltpu.VMEM((208,), f32),
            pltpu.VMEM((208,), f32),
            pltpu.VMEM((16,), f32),
            pltpu.SemaphoreType.DMA,
            pltpu.SemaphoreType.DMA,
            pltpu.SemaphoreType.DMA,
            pltpu.SemaphoreType.DMA,
            pltpu.SemaphoreType.DMA,
        ],
    )(wd_src, wd_dst, td_src, td_dst, word_ids, topic_ids,
      uws, uts, ones_hbm, zeros_hbm)

    loss, pred = pl.pallas_call(
        _k3_body,
        out_shape=[jax.ShapeDtypeStruct((1, 1), f32),
                   jax.ShapeDtypeStruct((1, B), f32)],
    )(partial.reshape(32, 16), y_data.reshape(1, B), b_wd.reshape(1, H),
      b_td.reshape(1, H), out_W, out_b.reshape(1, 1))

    return loss.reshape(()), pred.reshape(B, 1)


# speedup vs baseline: 1.0257x; 1.0186x over previous
import jax
import jax.numpy as jnp
from jax import lax
from jax.experimental import pallas as pl
from jax.experimental.pallas import tpu as pltpu
from jax.experimental.pallas import tpu_sc as plsc

N_WORD = 40000
N_TOPIC = 800
N_DOC = 3200
B = 16
H = 128
VOCAB = 15000
NUM_TOPIC = 50

OW = 40960
OT = 1024
ND = 3328
E_WD = 200000
E_TD = 20000
CW = 12544
CT = 1280
EW = 6272
ET = 640
NW_CH = 2560
NT_CH = 64


def _rsqrt16(x):
    i = lax.bitcast_convert_type(x, jnp.int32)
    i = jnp.int32(0x5F3759DF) - lax.shift_right_arithmetic(i, 1)
    y = lax.bitcast_convert_type(i, jnp.float32)
    for _ in range(2):
        y = y * (1.5 - 0.5 * x * y * y)
    return y


def _fill_i32(buf, start, count, value):
    v = jnp.full((16,), value, jnp.int32)

    def bd(i, _):
        buf[pl.ds(start + i * 16, 16)] = v
        return 0
    lax.fori_loop(0, count // 16, bd, 0)


def _k1_body(we_ref, te_ref, wwd_ref, wtd_ref, ow_ref, uws_ref, uts_ref):
    ow = ow_ref[...]
    vw = jnp.dot(wwd_ref[...], ow, preferred_element_type=jnp.float32)
    r = jnp.dot(we_ref[...], vw, preferred_element_type=jnp.float32)
    uws_ref[...] = jnp.broadcast_to(r, (r.shape[0], H))

    @pl.when(pl.program_id(0) == 0)
    def _():
        vt = jnp.dot(wtd_ref[...], ow, preferred_element_type=jnp.float32)
        rt = jnp.dot(te_ref[...], vt, preferred_element_type=jnp.float32)
        uts_ref[...] = jnp.broadcast_to(rt, (NUM_TOPIC, H))


def _sc_body(wd_src, wd_dst, td_src, td_dst, word_ids, topic_ids,
             uws, uts, ones_hbm, zeros_hbm, out,
             cnt_ow, cnt_iw, cnt_ot, cnt_it, m_wd, m_td, nv_w, nv_t,
             zbuf, ones, ones_t,
             ibufS, ibufD, tbufS, tbufD, jbufS, jbufD1, jbufD2, kbufS, kbufD,
             vbufA, vbufB, vbuf_t, gidx, gbuf, cbuf, nbuf,
             gidx_t, gbuf_t, cbuf_t, nbuf_t,
             mw, mt, ciw, cit, accbuf,
             semA, semB, semC, semD, semE):
    c = lax.axis_index("c")
    s = lax.axis_index("s")
    w = c * 16 + s

    d_ones = pltpu.async_copy(ones_hbm, ones, semA)
    d_onest = pltpu.async_copy(ones_hbm.at[pl.ds(0, CT)], ones_t, semA)
    d_zbuf = pltpu.async_copy(zeros_hbm, zbuf, semB)
    o_cw = jnp.where(s == 15, E_WD - CW, s * CW)
    d_is = pltpu.async_copy(wd_src.at[pl.ds(o_cw, CW)], ibufS, semC)
    d_id = pltpu.async_copy(wd_dst.at[pl.ds(o_cw, CW)], ibufD, semC)
    o_ct = jnp.where(s == 15, E_TD - CT, s * CT)
    d_ts = pltpu.async_copy(td_src.at[pl.ds(o_ct, CT)], tbufS, semC)
    d_td = pltpu.async_copy(td_dst.at[pl.ds(o_ct, CT)], tbufD, semC)
    o_bw = jnp.where(w == 31, E_WD - EW, w * EW)
    d_js = pltpu.async_copy(wd_src.at[pl.ds(o_bw, EW)], jbufS, semD)
    d_jd1 = pltpu.async_copy(wd_dst.at[pl.ds(o_bw, EW // 2)], jbufD1, semD)
    d_jd2 = pltpu.async_copy(wd_dst.at[pl.ds(o_bw + EW // 2, EW // 2)], jbufD2, semD)
    o_bt = jnp.where(w == 31, E_TD - ET, w * ET)
    d_ks = pltpu.async_copy(td_src.at[pl.ds(o_bt, ET)], kbufS, semD)
    d_kd = pltpu.async_copy(td_dst.at[pl.ds(o_bt, ET)], kbufD, semD)
    o_nw = jnp.where(s == 15, N_WORD - NW_CH, s * NW_CH)
    d_gi = pltpu.async_copy(word_ids.at[pl.ds(o_nw, NW_CH)], gidx, semE)
    o_nt = jnp.minimum(s * NT_CH, N_TOPIC - NT_CH)
    d_gt = pltpu.async_copy(topic_ids.at[pl.ds(o_nt, NT_CH)], gidx_t, semE)

    d_zbuf.wait()
    z1 = pltpu.async_copy(zbuf, cnt_ow.at[pl.ds(s * 2560, 2560)], semB)
    z2 = pltpu.async_copy(zbuf.at[pl.ds(0, 208)], cnt_iw.at[pl.ds(s * 208, 208)], semB)
    z3 = pltpu.async_copy(zbuf.at[pl.ds(0, 64)], cnt_ot.at[pl.ds(s * 64, 64)], semB)
    z4 = pltpu.async_copy(zbuf.at[pl.ds(0, 208)], cnt_it.at[pl.ds(s * 208, 208)], semB)
    z5 = pltpu.async_copy(zbuf.at[pl.ds(0, 208)], m_wd.at[pl.ds(s * 208, 208)], semB)
    z6 = pltpu.async_copy(zbuf.at[pl.ds(0, 208)], m_td.at[pl.ds(s * 208, 208)], semB)
    for z in (z1, z2, z3, z4, z5, z6):
        z.wait()

    d_gi.wait()
    d_gt.wait()

    def _sh16(i, _):
        gidx[pl.ds(i * 16, 16)] = lax.shift_left(gidx[pl.ds(i * 16, 16)], 7)
        return 0
    lax.fori_loop(0, NW_CH // 16, _sh16, 0)

    def _sht16(i, _):
        gidx_t[pl.ds(i * 16, 16)] = lax.shift_left(gidx_t[pl.ds(i * 16, 16)], 7)
        return 0
    lax.fori_loop(0, NT_CH // 16, _sht16, 0)
    d_uw = pltpu.async_copy(uws.at[gidx], gbuf, semE)
    d_ut = pltpu.async_copy(uts.at[gidx_t], gbuf_t, semE)
    plsc.subcore_barrier()

    for d in (d_is, d_id, d_ts, d_td, d_ones, d_onest):
        d.wait()

    @pl.when(s == 15)
    def _():
        _fill_i32(ibufS, 0, 704, OW - 1)
        _fill_i32(ibufD, 0, 704, ND - 1)
        _fill_i32(tbufS, 0, 480, OT - 1)
        _fill_i32(tbufD, 0, 480, ND - 1)

    s1 = pltpu.async_copy(ones, cnt_ow.at[ibufS], semC, add=True)
    s3 = pltpu.async_copy(ones_t, cnt_ot.at[tbufS], semC, add=True)
    s1.wait()
    s3.wait()
    plsc.subcore_barrier()

    s2 = pltpu.async_copy(ones, cnt_iw.at[ibufD], semC, add=True)
    s4 = pltpu.async_copy(ones_t, cnt_it.at[tbufD], semC, add=True)

    d_cb = pltpu.async_copy(cnt_ow.at[pl.ds(o_nw, NW_CH)], cbuf, semB)
    d_cbt = pltpu.async_copy(cnt_ot.at[pl.ds(o_nt, NT_CH)], cbuf_t, semB)
    d_uw.wait()
    d_ut.wait()
    d_cb.wait()
    d_cbt.wait()

    def _nv16(i, _):
        cnt = jnp.maximum(cbuf[pl.ds(i * 16, 16)], 1.0)
        nbuf[pl.ds(i * 16, 16)] = _rsqrt16(cnt) * gbuf[pl.ds(i * 16, 16)]
        return 0
    lax.fori_loop(0, NW_CH // 16, _nv16, 0)

    def _nvt16(i, _):
        cnt = jnp.maximum(cbuf_t[pl.ds(i * 16, 16)], 1.0)
        nbuf_t[pl.ds(i * 16, 16)] = _rsqrt16(cnt) * gbuf_t[pl.ds(i * 16, 16)]
        return 0
    lax.fori_loop(0, NT_CH // 16, _nvt16, 0)
    d_nv = pltpu.async_copy(nbuf, nv_w.at[pl.ds(o_nw, NW_CH)], semB)
    d_nvt = pltpu.async_copy(nbuf_t, nv_t.at[pl.ds(o_nt, NT_CH)], semB)
    d_nv.wait()
    d_nvt.wait()
    plsc.subcore_barrier()

    for d in (d_js, d_jd1, d_jd2, d_ks, d_kd):
        d.wait()

    @pl.when(w == 31)
    def _():
        _fill_i32(jbufS, 0, 704, OW - 1)
        _fill_i32(jbufD1, 0, 704, ND - 1)
        _fill_i32(kbufS, 0, 480, OT - 1)
        _fill_i32(kbufD, 0, 480, ND - 1)

    g1a = pltpu.async_copy(nv_w.at[jbufS.at[pl.ds(0, EW // 2)]], vbufA, semD)
    g1b = pltpu.async_copy(nv_w.at[jbufS.at[pl.ds(EW // 2, EW // 2)]], vbufB, semD)
    g2 = pltpu.async_copy(nv_t.at[kbufS], vbuf_t, semD)
    g1a.wait()
    x1a = pltpu.async_copy(vbufA, m_wd.at[jbufD1], semD, add=True)
    g1b.wait()
    x1b = pltpu.async_copy(vbufB, m_wd.at[jbufD2], semD, add=True)
    g2.wait()
    x2 = pltpu.async_copy(vbuf_t, m_td.at[kbufD], semD, add=True)
    x1a.wait()
    x1b.wait()
    x2.wait()
    s2.wait()
    s4.wait()
    plsc.subcore_barrier()

    base = s * 200
    r1 = pltpu.async_copy(m_wd.at[pl.ds(base, 208)], mw, semB)
    r2 = pltpu.async_copy(m_td.at[pl.ds(base, 208)], mt, semB)
    r3 = pltpu.async_copy(cnt_iw.at[pl.ds(base, 208)], ciw, semB)
    r4 = pltpu.async_copy(cnt_it.at[pl.ds(base, 208)], cit, semB)
    for d in (r1, r2, r3, r4):
        d.wait()
    lane = lax.iota(jnp.int32, 16)

    def _dot16(j, acc):
        cw = jnp.maximum(ciw[pl.ds(j * 16, 16)], 1.0)
        ct = jnp.maximum(cit[pl.ds(j * 16, 16)], 1.0)
        v = mw[pl.ds(j * 16, 16)] * _rsqrt16(cw) + mt[pl.ds(j * 16, 16)] * _rsqrt16(ct)
        return acc + jnp.where(j * 16 + lane < 200, v, 0.0)
    acc = lax.fori_loop(0, 13, _dot16, jnp.zeros((16,), jnp.float32))
    accbuf[...] = acc
    pltpu.sync_copy(accbuf, out.at[pl.ds(w * 16, 16)])


def _k3_body(p_ref, y_ref, bw_ref, bt_ref, ow_ref, ob_ref, loss_ref, pred_ref):
    dsum = jnp.sum(p_ref[...], axis=0, keepdims=True)
    bias = jnp.sum((bw_ref[...] + bt_ref[...]) * ow_ref[...].T) + ob_ref[0, 0]
    logits = dsum / 200.0 + bias
    y = y_ref[...]
    loss_ref[...] = jnp.mean(
        jnp.maximum(logits, 0.0) - logits * y
        + jnp.log(1.0 + jnp.exp(-jnp.abs(logits))), keepdims=True).reshape(1, 1)
    pred_ref[...] = 1.0 / (1.0 + jnp.exp(-logits))


def kernel(word_ids, topic_ids, wd_src, wd_dst, ww_src, ww_dst, wt_src, wt_dst,
           td_src, td_dst, tt_src, tt_dst, y_data, word_embeds, topic_embeds,
           W_wt, b_wt, W_ww, b_ww, W_wd, b_wd, W_td, b_td, W_tt, b_tt, out_W, out_b):
    f32, i32 = jnp.float32, jnp.int32

    uws, uts = pl.pallas_call(
        _k1_body,
        grid=(5,),
        in_specs=[pl.BlockSpec((3072, H), lambda i: (i, 0)),
                  pl.BlockSpec((NUM_TOPIC, H), lambda i: (0, 0)),
                  pl.BlockSpec((H, H), lambda i: (0, 0)),
                  pl.BlockSpec((H, H), lambda i: (0, 0)),
                  pl.BlockSpec((H, 1), lambda i: (0, 0))],
        out_specs=[pl.BlockSpec((3072, H), lambda i: (i, 0)),
                   pl.BlockSpec((NUM_TOPIC, H), lambda i: (0, 0))],
        out_shape=[jax.ShapeDtypeStruct((VOCAB, H), f32),
                   jax.ShapeDtypeStruct((NUM_TOPIC, H), f32)],
    )(word_embeds, topic_embeds, W_wd, W_td, out_W)
    uws = uws.reshape(VOCAB * H)
    uts = uts.reshape(NUM_TOPIC * H)

    ones_hbm = jnp.ones((CW,), f32)
    zeros_hbm = jnp.zeros((2560,), f32)

    mesh = plsc.VectorSubcoreMesh(core_axis_name="c", subcore_axis_name="s")
    partial = pl.kernel(
        _sc_body,
        out_type=jax.ShapeDtypeStruct((512,), f32),
        mesh=mesh,
        scratch_types=[
            pltpu.VMEM_SHARED((OW,), f32),
            pltpu.VMEM_SHARED((ND,), f32),
            pltpu.VMEM_SHARED((OT,), f32),
            pltpu.VMEM_SHARED((ND,), f32),
            pltpu.VMEM_SHARED((ND,), f32),
            pltpu.VMEM_SHARED((ND,), f32),
            pltpu.VMEM_SHARED((OW,), f32),
            pltpu.VMEM_SHARED((OT,), f32),
            pltpu.VMEM((2560,), f32),
            pltpu.VMEM((CW,), f32),
            pltpu.VMEM((CT,), f32),
            pltpu.VMEM((CW,), i32),
            pltpu.VMEM((CW,), i32),
            pltpu.VMEM((CT,), i32),
            pltpu.VMEM((CT,), i32),
            pltpu.VMEM((EW,), i32),
            pltpu.VMEM((EW // 2,), i32),
            pltpu.VMEM((EW // 2,), i32),
            pltpu.VMEM((ET,), i32),
            pltpu.VMEM((ET,), i32),
            pltpu.VMEM((EW // 2,), f32),
            pltpu.VMEM((EW // 2,), f32),
            pltpu.VMEM((ET,), f32),
            pltpu.VMEM((NW_CH,), i32),
            pltpu.VMEM((NW_CH,), f32),
            pltpu.VMEM((NW_CH,), f32),
            pltpu.VMEM((NW_CH,), f32),
            pltpu.VMEM((NT_CH,), i32),
            pltpu.VMEM((NT_CH,), f32),
            pltpu.VMEM((NT_CH,), f32),
            pltpu.VMEM((NT_CH,), f32),
            pltpu.VMEM((208,), f32),
            pltpu.VMEM((208,), f32),
            pltpu.VMEM((208,), f32),
            pltpu.VMEM((208,), f32),
            pltpu.VMEM((16,), f32),
            pltpu.SemaphoreType.DMA,
            pltpu.SemaphoreType.DMA,
            pltpu.SemaphoreType.DMA,
            pltpu.SemaphoreType.DMA,
            pltpu.SemaphoreType.DMA,
        ],
    )(wd_src, wd_dst, td_src, td_dst, word_ids, topic_ids,
      uws, uts, ones_hbm, zeros_hbm)

    loss, pred = pl.pallas_call(
        _k3_body,
        out_shape=[jax.ShapeDtypeStruct((1, 1), f32),
                   jax.ShapeDtypeStruct((1, B), f32)],
    )(partial.reshape(32, 16), y_data.reshape(1, B), b_wd.reshape(1, H),
      b_td.reshape(1, H), out_W, out_b.reshape(1, 1))

    return loss.reshape(()), pred.reshape(B, 1)
